# baseline (speedup 1.0000x reference)
import functools
import math

import jax
import jax.numpy as jnp
from jax import lax
from jax.experimental import pallas as pl
from jax.experimental.pallas import tpu as pltpu
from jax.experimental.pallas import tpu_sc as plsc

NC = 2
NS = 16
NW = NC * NS
LN = 16
K = 6
HALF = 16

_mesh = lambda: plsc.VectorSubcoreMesh(core_axis_name="c", subcore_axis_name="s")


def _deg_kernel(ep, np_):
    rows_per_w = (ep // 128) // NW
    nch = rows_per_w // K
    chunk = K * 128

    @functools.partial(
        pl.kernel,
        out_type=jax.ShapeDtypeStruct((NW, np_), jnp.float32),
        mesh=_mesh(),
        scratch_types=[
            pltpu.VMEM((2, chunk), jnp.int32),
            pltpu.VMEM((np_,), jnp.float32),
            pltpu.SemaphoreType.DMA,
        ],
        compiler_params=pltpu.CompilerParams(needs_layout_passes=False, use_tc_tiling_on_sc=False),
    )
    def k(dst_hbm, partials_hbm, idx_v, deg_l, isem):
        c = lax.axis_index("c")
        s = lax.axis_index("s")
        w = c * NS + s
        zeros = jnp.zeros((LN,), jnp.float32)
        ones = jnp.ones((LN,), jnp.float32)

        @pl.loop(0, np_ // LN, unroll=8)
        def _zero(i):
            deg_l[pl.ds(i * LN, LN)] = zeros

        base = w * rows_per_w * 128

        def load(ch, p):
            return pltpu.async_copy(
                dst_hbm.at[pl.ds(base + ch * chunk, chunk)], idx_v.at[p], isem)

        load(0, 0).wait()

        @pl.loop(0, nch)
        def _chunk(ch):
            p = lax.rem(ch, 2)
            nxt = pltpu.async_copy(
                dst_hbm.at[pl.ds(base + lax.min(ch + 1, nch - 1) * chunk, chunk)],
                idx_v.at[1 - p], isem)

            @pl.loop(0, chunk // LN, unroll=8)
            def _vec(j):
                idx = idx_v[p, pl.ds(j * LN, LN)]
                plsc.addupdate_scatter(deg_l, [idx], ones)

            nxt.wait()

        pltpu.sync_copy(deg_l, partials_hbm.at[w])

    return k


def _agg_kernel(ep, np_, split_edges):
    total_rows = ep // 128
    rows_per_tile = total_rows // (NW if split_edges else NS)
    nch = rows_per_tile // K
    nrl = np_ // NS

    @functools.partial(
        pl.kernel,
        out_type=(
            jax.ShapeDtypeStruct((np_, HALF), jnp.float32),
            jax.ShapeDtypeStruct((np_, HALF), jnp.float32),
        ),
        mesh=_mesh(),
        scratch_types=[
            pltpu.VMEM((2, K, 128), jnp.int32),
            pltpu.VMEM((2, K, 128), jnp.int32),
            pltpu.VMEM((2, K, 128, HALF), jnp.float32),
            pltpu.VMEM_SHARED((np_, HALF), jnp.float32),
            pltpu.SemaphoreType.DMA,
            pltpu.SemaphoreType.DMA,
        ],
        compiler_params=pltpu.CompilerParams(needs_layout_passes=False, use_tc_tiling_on_sc=False),
    )
    def k(ta, tb, src_hbm, dst_hbm, oa, ob, sidx, didx, rows, acc, gsem, ssem):
        c = lax.axis_index("c")
        s = lax.axis_index("s")
        rb = s * nrl
        if split_edges:
            base = (c * NS + s) * rows_per_tile
        else:
            base = s * rows_per_tile

        def run(table, out):
            pltpu.sync_copy(table.at[pl.ds(rb, nrl)], acc.at[pl.ds(rb, nrl)])
            plsc.subcore_barrier()

            def load_idx(ch, p):
                r0 = base + ch * K
                pltpu.sync_copy(src_hbm.at[pl.ds(r0, K)], sidx.at[p])
                pltpu.sync_copy(dst_hbm.at[pl.ds(r0, K)], didx.at[p])

            def fire_gather(p):
                return [
                    pltpu.async_copy(table.at[sidx.at[p, j]], rows.at[p, j], gsem)
                    for j in range(K)
                ]

            def fire_scatter(p):
                return [
                    pltpu.async_copy(rows.at[p, j], acc.at[didx.at[p, j]], ssem,
                                     add=True)
                    for j in range(K)
                ]

            def drain_scatter(p):
                for j in range(K):
                    pltpu.make_async_copy(rows.at[p, j], acc.at[didx.at[p, j]],
                                          ssem).wait()

            npair = nch // 2
            tail = nch - 2 * npair

            @pl.loop(0, npair)
            def _pair(chp):
                load_idx(2 * chp, 0)
                ga = fire_gather(0)

                @pl.when(chp > 0)
                def _():
                    drain_scatter(1)

                load_idx(2 * chp + 1, 1)
                for d in ga:
                    d.wait()
                fire_scatter(0)
                gb = fire_gather(1)
                drain_scatter(0)
                for d in gb:
                    d.wait()
                fire_scatter(1)

            if npair > 0:
                drain_scatter(1)

            if tail:
                load_idx(nch - 1, 0)
                for d in fire_gather(0):
                    d.wait()
                fire_scatter(0)
                drain_scatter(0)

            plsc.subcore_barrier()
            pltpu.sync_copy(acc.at[pl.ds(rb, nrl)], out.at[pl.ds(rb, nrl)])

        pl.when(c == 0)(lambda: run(ta, oa))
        pl.when(c == 1)(lambda: run(tb, ob))

    return k


def _softplus(v):
    return jnp.log1p(jnp.exp(v))


def _weights_kl_body(w1_mu, w1_rho, ew1, b1_mu, b1_rho, eb1,
                     w2_mu, w2_rho, ew2, b2_mu, b2_rho, eb2,
                     w1f, b1f, w2f, b2f, kl):
    def sample_kl(mu, rho, eps):
        sig = _softplus(rho[...])
        w = mu[...] + sig * eps[...]
        klv = jnp.sum(-jnp.log(sig) + 0.5 * (sig * sig + mu[...] * mu[...]) - 0.5)
        return w, klv

    w1, k1 = sample_kl(w1_mu, w1_rho, ew1)
    b1, k2 = sample_kl(b1_mu, b1_rho, eb1)
    w2, k3 = sample_kl(w2_mu, w2_rho, ew2)
    b2, k4 = sample_kl(b2_mu, b2_rho, eb2)
    w1f[...] = w1
    b1f[...] = b1
    w2f[...] = w2
    b2f[...] = b2
    kl[...] = (k1 + k2 + k3 + k4).reshape(1, 1)


def _finalize_kernel(np_, in_f):
    nt = np_ // NW
    ch = 448
    nchk = nt // ch

    @functools.partial(
        pl.kernel,
        out_type=(
            jax.ShapeDtypeStruct((np_, HALF), jnp.float32),
            jax.ShapeDtypeStruct((np_, HALF), jnp.float32),
            jax.ShapeDtypeStruct((np_, HALF), jnp.float32),
        ),
        mesh=_mesh(),
        scratch_types=[
            pltpu.VMEM((NW, ch), jnp.float32),
            pltpu.VMEM((ch, in_f), jnp.float32),
            pltpu.VMEM((ch,), jnp.float32),
            pltpu.VMEM((ch, HALF), jnp.float32),
            pltpu.VMEM((ch, HALF), jnp.float32),
            pltpu.VMEM((ch, HALF), jnp.float32),
            pltpu.SemaphoreType.DMA,
            pltpu.SemaphoreType.DMA,
        ],
        compiler_params=pltpu.CompilerParams(needs_layout_passes=False, use_tc_tiling_on_sc=False),
    )
    def k(part_hbm, x_hbm, dinvr_hbm, p1a_hbm, p1b_hbm,
          part_v, x_v, dinv_v, dr_v, pa_v, pb_v, sem, xsem):
        c = lax.axis_index("c")
        s = lax.axis_index("s")
        w = c * NS + s
        ones = jnp.ones((LN,), jnp.float32)
        magic = jnp.int32(0x5F3759DF)

        @pl.loop(0, nchk)
        def _chunk(ci):
            off = w * nt + ci * ch
            xd = pltpu.async_copy(x_hbm.at[pl.ds(off, ch)], x_v, xsem)
            pd = [
                pltpu.async_copy(part_hbm.at[pl.ds(w2 * np_ + off, ch)],
                                 part_v.at[w2], sem)
                for w2 in range(NW)
            ]
            for d in pd:
                d.wait()

            @pl.loop(0, ch // LN)
            def _rsqrt(j):
                d = ones
                for w2 in range(NW):
                    d = d + part_v[w2, pl.ds(j * LN, LN)]
                y = plsc.bitcast(magic - (plsc.bitcast(d, jnp.int32) >> 1),
                                 jnp.float32)
                for _ in range(3):
                    y = y * (1.5 - 0.5 * d * y * y)
                dinv_v[pl.ds(j * LN, LN)] = y

            xd.wait()

            @pl.loop(0, ch // LN)
            def _expand(j):
                dv16 = dinv_v[pl.ds(j * LN, LN)]
                for t in range(LN):
                    i = j * LN + t
                    dval = dv16[t]
                    dr_v[i, pl.ds(0, HALF)] = ones * dval
                    pa_v[i, pl.ds(0, HALF)] = x_v[i, pl.ds(0, HALF)] * dval
                    pb_v[i, pl.ds(0, HALF)] = x_v[i, pl.ds(HALF, HALF)] * dval

            pltpu.sync_copy(dr_v, dinvr_hbm.at[pl.ds(off, ch)])
            pltpu.sync_copy(pa_v, p1a_hbm.at[pl.ds(off, ch)])
            pltpu.sync_copy(pb_v, p1b_hbm.at[pl.ds(off, ch)])

    return k


def _mid_body(aa_ref, ab_ref, dr_ref, w1a_ref, w1b_ref, w2_ref, b1_ref,
              p2_ref):
    dr = dr_ref[...]
    h = (jnp.dot(aa_ref[...] * dr, w1a_ref[...], preferred_element_type=jnp.float32)
         + jnp.dot(ab_ref[...] * dr, w1b_ref[...], preferred_element_type=jnp.float32)
         + b1_ref[...])
    h = jnp.maximum(h, 0.0)
    p2_ref[...] = jnp.dot(h, w2_ref[...], preferred_element_type=jnp.float32) * dr


def _fin_body(aa_ref, ab_ref, p2_ref, dr_ref, b2_ref, out_ref):
    agg = aa_ref[...] + ab_ref[...] - p2_ref[...]
    out_ref[...] = agg * dr_ref[...] + b2_ref[...]


def kernel(x, edge_index, w1_mu, w1_rho, b1_mu, b1_rho,
           w2_mu, w2_rho, b2_mu, b2_rho):
    n, in_f = x.shape
    e = edge_index.shape[1]
    hid = w1_mu.shape[1]
    out_f = w2_mu.shape[1]
    f32 = jnp.float32

    np_ = ((n + 127) // 128 + 2) * 128
    emult = NW * 128 * K
    ep = ((e + emult - 1) // emult) * emult

    ei = edge_index.astype(jnp.int32)
    pad_e = ep - e
    pad_src = jnp.full((pad_e,), n, jnp.int32)
    pad_dst = n + (jnp.arange(pad_e, dtype=jnp.int32) % (np_ - n))
    src2d = jnp.concatenate([ei[0], pad_src]).reshape(ep // 128, 128)
    dst2d = jnp.concatenate([ei[1], pad_dst]).reshape(ep // 128, 128)
    dst1d = dst2d.reshape(ep)
    x_pad = jnp.pad(x, ((0, np_ - n), (0, 0)))

    kk = jax.random.key(42)
    k1, k2, k3, k4 = jax.random.split(kk, 4)
    eps_w1 = jax.random.normal(k1, w1_mu.shape, dtype=w1_mu.dtype)
    eps_b1 = jax.random.normal(k2, b1_mu.shape, dtype=b1_mu.dtype)
    eps_w2 = jax.random.normal(k3, w2_mu.shape, dtype=w2_mu.dtype)
    eps_b2 = jax.random.normal(k4, b2_mu.shape, dtype=b2_mu.dtype)

    rho_pad = math.log(math.e - 1.0)
    padw = HALF - out_f
    w2_mu_p = jnp.pad(w2_mu, ((0, 0), (0, padw)))
    w2_rho_p = jnp.pad(w2_rho, ((0, 0), (0, padw)), constant_values=rho_pad)
    eps_w2_p = jnp.pad(eps_w2, ((0, 0), (0, padw)))
    b2_mu_p = jnp.pad(b2_mu, (0, padw)).reshape(1, HALF)
    b2_rho_p = jnp.pad(b2_rho, (0, padw), constant_values=rho_pad).reshape(1, HALF)
    eps_b2_p = jnp.pad(eps_b2, (0, padw)).reshape(1, HALF)
    b1_mu_r = b1_mu.reshape(1, hid)
    b1_rho_r = b1_rho.reshape(1, hid)
    eps_b1_r = eps_b1.reshape(1, hid)

    w1f, b1f, w2f, b2f, kl = pl.pallas_call(
        _weights_kl_body,
        out_shape=(
            jax.ShapeDtypeStruct((in_f, hid), f32),
            jax.ShapeDtypeStruct((1, hid), f32),
            jax.ShapeDtypeStruct((hid, HALF), f32),
            jax.ShapeDtypeStruct((1, HALF), f32),
            jax.ShapeDtypeStruct((1, 1), f32),
        ),
    )(w1_mu, w1_rho, eps_w1, b1_mu_r, b1_rho_r, eps_b1_r,
      w2_mu_p, w2_rho_p, eps_w2_p, b2_mu_p, b2_rho_p, eps_b2_p)

    r128 = jnp.arange(128)
    c512 = jnp.arange(512)
    mask_a = (r128[:, None] // HALF) == (c512[None, :] // hid)
    w1a_big = jnp.where(mask_a, w1f[:HALF][r128 % HALF][:, c512 % hid], 0.0)
    w1b_big = jnp.where(mask_a, w1f[HALF:][r128 % HALF][:, c512 % hid], 0.0)
    mask_2 = (c512[:, None] // hid) == (r128[None, :] // HALF)
    w2_big = jnp.where(mask_2, w2f[c512 % hid][:, r128 % HALF], 0.0)
    b1t = jnp.tile(b1f, (1, 8))
    b2t = jnp.tile(b2f, (1, 8))

    partials = _deg_kernel(ep, np_)(dst1d)

    dinvr, p1a, p1b = _finalize_kernel(np_, in_f)(partials.reshape(NW * np_), x_pad)

    agg1a, agg1b = _agg_kernel(ep, np_, split_edges=False)(p1a, p1b, src2d, dst2d)

    npk = np_ // 8
    pk = lambda a: a.reshape(npk, 128)
    nb = npk // 8
    grid = npk // nb
    blk = pl.BlockSpec((nb, 128), lambda i: (i, 0))
    p2p = pl.pallas_call(
        _mid_body,
        grid=(grid,),
        in_specs=[
            blk, blk, blk,
            pl.BlockSpec((128, 512), lambda i: (0, 0)),
            pl.BlockSpec((128, 512), lambda i: (0, 0)),
            pl.BlockSpec((512, 128), lambda i: (0, 0)),
            pl.BlockSpec((1, 512), lambda i: (0, 0)),
        ],
        out_specs=blk,
        out_shape=jax.ShapeDtypeStruct((npk, 128), f32),
    )(pk(agg1a), pk(agg1b), pk(dinvr), w1a_big, w1b_big, w2_big, b1t)
    p2 = p2p.reshape(np_, HALF)

    agg2a, agg2b = _agg_kernel(ep, np_, split_edges=True)(p2, p2, src2d, dst2d)

    outp = pl.pallas_call(
        _fin_body,
        grid=(grid,),
        in_specs=[blk, blk, blk, blk, pl.BlockSpec((1, 128), lambda i: (0, 0))],
        out_specs=blk,
        out_shape=jax.ShapeDtypeStruct((npk, 128), f32),
    )(pk(agg2a), pk(agg2b), p2p, pk(dinvr), b2t)

    return outp[: n // 8].reshape(n, HALF)[:, :out_f], kl[0, 0]


# speedup vs baseline: 1.1034x; 1.1034x over previous
import functools
import math

import jax
import jax.numpy as jnp
from jax import lax
from jax.experimental import pallas as pl
from jax.experimental.pallas import tpu as pltpu
from jax.experimental.pallas import tpu_sc as plsc

NC = 2
NS = 16
NW = NC * NS
LN = 16
K = 4
HALF = 16

_mesh = lambda: plsc.VectorSubcoreMesh(core_axis_name="c", subcore_axis_name="s")


def _deg_kernel(ep, np_):
    rows_per_w = (ep // 128) // NW
    nch = rows_per_w // K
    chunk = K * 128

    @functools.partial(
        pl.kernel,
        out_type=jax.ShapeDtypeStruct((NW, np_), jnp.float32),
        mesh=_mesh(),
        scratch_types=[
            pltpu.VMEM((2, chunk), jnp.int32),
            pltpu.VMEM((np_,), jnp.float32),
            pltpu.SemaphoreType.DMA,
        ],
        compiler_params=pltpu.CompilerParams(needs_layout_passes=False, use_tc_tiling_on_sc=False),
    )
    def k(dst_hbm, partials_hbm, idx_v, deg_l, isem):
        c = lax.axis_index("c")
        s = lax.axis_index("s")
        w = c * NS + s
        zeros = jnp.zeros((LN,), jnp.float32)
        ones = jnp.ones((LN,), jnp.float32)

        @pl.loop(0, np_ // LN, unroll=8)
        def _zero(i):
            deg_l[pl.ds(i * LN, LN)] = zeros

        base = w * rows_per_w * 128

        def load(ch, p):
            return pltpu.async_copy(
                dst_hbm.at[pl.ds(base + ch * chunk, chunk)], idx_v.at[p], isem)

        load(0, 0).wait()

        @pl.loop(0, nch)
        def _chunk(ch):
            p = lax.rem(ch, 2)
            nxt = pltpu.async_copy(
                dst_hbm.at[pl.ds(base + lax.min(ch + 1, nch - 1) * chunk, chunk)],
                idx_v.at[1 - p], isem)

            @pl.loop(0, chunk // LN, unroll=8)
            def _vec(j):
                idx = idx_v[p, pl.ds(j * LN, LN)]
                plsc.addupdate_scatter(deg_l, [idx], ones)

            nxt.wait()

        pltpu.sync_copy(deg_l, partials_hbm.at[w])

    return k


def _agg_kernel(ep, np_, split_edges):
    total_rows = ep // 128
    rows_per_tile = total_rows // (NW if split_edges else NS)
    nch = rows_per_tile // K
    nrl = np_ // NS

    @functools.partial(
        pl.kernel,
        out_type=(
            jax.ShapeDtypeStruct((np_, HALF), jnp.float32),
            jax.ShapeDtypeStruct((np_, HALF), jnp.float32),
        ),
        mesh=_mesh(),
        scratch_types=[
            pltpu.VMEM((2, K, 128), jnp.int32),
            pltpu.VMEM((2, K, 128), jnp.int32),
            pltpu.VMEM((2, K, 128, HALF), jnp.float32),
            pltpu.VMEM_SHARED((np_, HALF), jnp.float32),
            pltpu.SemaphoreType.DMA,
            pltpu.SemaphoreType.DMA,
        ],
        compiler_params=pltpu.CompilerParams(needs_layout_passes=False, use_tc_tiling_on_sc=False),
    )
    def k(ta, tb, src_hbm, dst_hbm, oa, ob, sidx, didx, rows, acc, gsem, ssem):
        c = lax.axis_index("c")
        s = lax.axis_index("s")
        rb = s * nrl
        if split_edges:
            base = (c * NS + s) * rows_per_tile
        else:
            base = s * rows_per_tile

        def run(table, out):
            pltpu.sync_copy(table.at[pl.ds(rb, nrl)], acc.at[pl.ds(rb, nrl)])
            plsc.subcore_barrier()

            def load_idx(ch, p):
                r0 = base + ch * K
                pltpu.sync_copy(src_hbm.at[pl.ds(r0, K)], sidx.at[p])
                pltpu.sync_copy(dst_hbm.at[pl.ds(r0, K)], didx.at[p])

            def fire_gather(p):
                return [
                    pltpu.async_copy(table.at[sidx.at[p, j]], rows.at[p, j], gsem)
                    for j in range(K)
                ]

            def fire_scatter(p):
                return [
                    pltpu.async_copy(rows.at[p, j], acc.at[didx.at[p, j]], ssem,
                                     add=True)
                    for j in range(K)
                ]

            def drain_scatter(p):
                for j in range(K):
                    pltpu.make_async_copy(rows.at[p, j], acc.at[didx.at[p, j]],
                                          ssem).wait()

            npair = nch // 2
            tail = nch - 2 * npair

            @pl.loop(0, npair)
            def _pair(chp):
                load_idx(2 * chp, 0)
                ga = fire_gather(0)

                @pl.when(chp > 0)
                def _():
                    drain_scatter(1)

                load_idx(2 * chp + 1, 1)
                for d in ga:
                    d.wait()
                fire_scatter(0)
                gb = fire_gather(1)
                drain_scatter(0)
                for d in gb:
                    d.wait()
                fire_scatter(1)

            if npair > 0:
                drain_scatter(1)

            if tail:
                load_idx(nch - 1, 0)
                for d in fire_gather(0):
                    d.wait()
                fire_scatter(0)
                drain_scatter(0)

            plsc.subcore_barrier()
            pltpu.sync_copy(acc.at[pl.ds(rb, nrl)], out.at[pl.ds(rb, nrl)])

        pl.when(c == 0)(lambda: run(ta, oa))
        pl.when(c == 1)(lambda: run(tb, ob))

    return k


def _softplus(v):
    return jnp.log1p(jnp.exp(v))


def _weights_kl_body(w1_mu, w1_rho, ew1, b1_mu, b1_rho, eb1,
                     w2_mu, w2_rho, ew2, b2_mu, b2_rho, eb2,
                     w1f, b1f, w2f, b2f, kl):
    def sample_kl(mu, rho, eps):
        sig = _softplus(rho[...])
        w = mu[...] + sig * eps[...]
        klv = jnp.sum(-jnp.log(sig) + 0.5 * (sig * sig + mu[...] * mu[...]) - 0.5)
        return w, klv

    w1, k1 = sample_kl(w1_mu, w1_rho, ew1)
    b1, k2 = sample_kl(b1_mu, b1_rho, eb1)
    w2, k3 = sample_kl(w2_mu, w2_rho, ew2)
    b2, k4 = sample_kl(b2_mu, b2_rho, eb2)
    w1f[...] = w1
    b1f[...] = b1
    w2f[...] = w2
    b2f[...] = b2
    kl[...] = (k1 + k2 + k3 + k4).reshape(1, 1)


def _finalize_kernel(np_, in_f):
    nt = np_ // NW
    ch = 448
    nchk = nt // ch

    @functools.partial(
        pl.kernel,
        out_type=(
            jax.ShapeDtypeStruct((np_, HALF), jnp.float32),
            jax.ShapeDtypeStruct((np_, HALF), jnp.float32),
            jax.ShapeDtypeStruct((np_, HALF), jnp.float32),
        ),
        mesh=_mesh(),
        scratch_types=[
            pltpu.VMEM((NW, ch), jnp.float32),
            pltpu.VMEM((ch, in_f), jnp.float32),
            pltpu.VMEM((ch,), jnp.float32),
            pltpu.VMEM((ch, HALF), jnp.float32),
            pltpu.VMEM((ch, HALF), jnp.float32),
            pltpu.VMEM((ch, HALF), jnp.float32),
            pltpu.SemaphoreType.DMA,
            pltpu.SemaphoreType.DMA,
        ],
        compiler_params=pltpu.CompilerParams(needs_layout_passes=False, use_tc_tiling_on_sc=False),
    )
    def k(part_hbm, x_hbm, dinvr_hbm, p1a_hbm, p1b_hbm,
          part_v, x_v, dinv_v, dr_v, pa_v, pb_v, sem, xsem):
        c = lax.axis_index("c")
        s = lax.axis_index("s")
        w = c * NS + s
        ones = jnp.ones((LN,), jnp.float32)
        magic = jnp.int32(0x5F3759DF)

        @pl.loop(0, nchk)
        def _chunk(ci):
            off = w * nt + ci * ch
            xd = pltpu.async_copy(x_hbm.at[pl.ds(off, ch)], x_v, xsem)
            pd = [
                pltpu.async_copy(part_hbm.at[pl.ds(w2 * np_ + off, ch)],
                                 part_v.at[w2], sem)
                for w2 in range(NW)
            ]
            for d in pd:
                d.wait()

            @pl.loop(0, ch // LN)
            def _rsqrt(j):
                d = ones
                for w2 in range(NW):
                    d = d + part_v[w2, pl.ds(j * LN, LN)]
                y = plsc.bitcast(magic - (plsc.bitcast(d, jnp.int32) >> 1),
                                 jnp.float32)
                for _ in range(3):
                    y = y * (1.5 - 0.5 * d * y * y)
                dinv_v[pl.ds(j * LN, LN)] = y

            xd.wait()

            @pl.loop(0, ch // LN)
            def _expand(j):
                dv16 = dinv_v[pl.ds(j * LN, LN)]
                for t in range(LN):
                    i = j * LN + t
                    dval = dv16[t]
                    dr_v[i, pl.ds(0, HALF)] = ones * dval
                    pa_v[i, pl.ds(0, HALF)] = x_v[i, pl.ds(0, HALF)] * dval
                    pb_v[i, pl.ds(0, HALF)] = x_v[i, pl.ds(HALF, HALF)] * dval

            pltpu.sync_copy(dr_v, dinvr_hbm.at[pl.ds(off, ch)])
            pltpu.sync_copy(pa_v, p1a_hbm.at[pl.ds(off, ch)])
            pltpu.sync_copy(pb_v, p1b_hbm.at[pl.ds(off, ch)])

    return k


def _mid_body(aa_ref, ab_ref, dr_ref, w1a_ref, w1b_ref, w2_ref, b1_ref,
              p2_ref):
    dr = dr_ref[...]
    h = (jnp.dot(aa_ref[...] * dr, w1a_ref[...], preferred_element_type=jnp.float32)
         + jnp.dot(ab_ref[...] * dr, w1b_ref[...], preferred_element_type=jnp.float32)
         + b1_ref[...])
    h = jnp.maximum(h, 0.0)
    p2_ref[...] = jnp.dot(h, w2_ref[...], preferred_element_type=jnp.float32) * dr


def _fin_body(aa_ref, ab_ref, p2_ref, dr_ref, b2_ref, out_ref):
    agg = aa_ref[...] + ab_ref[...] - p2_ref[...]
    out_ref[...] = agg * dr_ref[...] + b2_ref[...]


def kernel(x, edge_index, w1_mu, w1_rho, b1_mu, b1_rho,
           w2_mu, w2_rho, b2_mu, b2_rho):
    n, in_f = x.shape
    e = edge_index.shape[1]
    hid = w1_mu.shape[1]
    out_f = w2_mu.shape[1]
    f32 = jnp.float32

    np_ = ((n + 127) // 128 + 2) * 128
    emult = NW * 128 * K
    ep = ((e + emult - 1) // emult) * emult

    ei = edge_index.astype(jnp.int32)
    pad_e = ep - e
    pad_src = jnp.full((pad_e,), n, jnp.int32)
    pad_dst = n + (jnp.arange(pad_e, dtype=jnp.int32) % (np_ - n))
    src2d = jnp.concatenate([ei[0], pad_src]).reshape(ep // 128, 128)
    dst2d = jnp.concatenate([ei[1], pad_dst]).reshape(ep // 128, 128)
    dst1d = dst2d.reshape(ep)
    x_pad = jnp.pad(x, ((0, np_ - n), (0, 0)))

    kk = jax.random.key(42)
    k1, k2, k3, k4 = jax.random.split(kk, 4)
    eps_w1 = jax.random.normal(k1, w1_mu.shape, dtype=w1_mu.dtype)
    eps_b1 = jax.random.normal(k2, b1_mu.shape, dtype=b1_mu.dtype)
    eps_w2 = jax.random.normal(k3, w2_mu.shape, dtype=w2_mu.dtype)
    eps_b2 = jax.random.normal(k4, b2_mu.shape, dtype=b2_mu.dtype)

    rho_pad = math.log(math.e - 1.0)
    padw = HALF - out_f
    w2_mu_p = jnp.pad(w2_mu, ((0, 0), (0, padw)))
    w2_rho_p = jnp.pad(w2_rho, ((0, 0), (0, padw)), constant_values=rho_pad)
    eps_w2_p = jnp.pad(eps_w2, ((0, 0), (0, padw)))
    b2_mu_p = jnp.pad(b2_mu, (0, padw)).reshape(1, HALF)
    b2_rho_p = jnp.pad(b2_rho, (0, padw), constant_values=rho_pad).reshape(1, HALF)
    eps_b2_p = jnp.pad(eps_b2, (0, padw)).reshape(1, HALF)
    b1_mu_r = b1_mu.reshape(1, hid)
    b1_rho_r = b1_rho.reshape(1, hid)
    eps_b1_r = eps_b1.reshape(1, hid)

    w1f, b1f, w2f, b2f, kl = pl.pallas_call(
        _weights_kl_body,
        out_shape=(
            jax.ShapeDtypeStruct((in_f, hid), f32),
            jax.ShapeDtypeStruct((1, hid), f32),
            jax.ShapeDtypeStruct((hid, HALF), f32),
            jax.ShapeDtypeStruct((1, HALF), f32),
            jax.ShapeDtypeStruct((1, 1), f32),
        ),
    )(w1_mu, w1_rho, eps_w1, b1_mu_r, b1_rho_r, eps_b1_r,
      w2_mu_p, w2_rho_p, eps_w2_p, b2_mu_p, b2_rho_p, eps_b2_p)

    r128 = jnp.arange(128)
    c512 = jnp.arange(512)
    mask_a = (r128[:, None] // HALF) == (c512[None, :] // hid)
    w1a_big = jnp.where(mask_a, w1f[:HALF][r128 % HALF][:, c512 % hid], 0.0)
    w1b_big = jnp.where(mask_a, w1f[HALF:][r128 % HALF][:, c512 % hid], 0.0)
    mask_2 = (c512[:, None] // hid) == (r128[None, :] // HALF)
    w2_big = jnp.where(mask_2, w2f[c512 % hid][:, r128 % HALF], 0.0)
    b1t = jnp.tile(b1f, (1, 8))
    b2t = jnp.tile(b2f, (1, 8))

    partials = _deg_kernel(ep, np_)(dst1d)

    dinvr, p1a, p1b = _finalize_kernel(np_, in_f)(partials.reshape(NW * np_), x_pad)

    agg1a, agg1b = _agg_kernel(ep, np_, split_edges=False)(p1a, p1b, src2d, dst2d)

    npk = np_ // 8
    pk = lambda a: a.reshape(npk, 128)
    nb = npk // 8
    grid = npk // nb
    blk = pl.BlockSpec((nb, 128), lambda i: (i, 0))
    p2p = pl.pallas_call(
        _mid_body,
        grid=(grid,),
        in_specs=[
            blk, blk, blk,
            pl.BlockSpec((128, 512), lambda i: (0, 0)),
            pl.BlockSpec((128, 512), lambda i: (0, 0)),
            pl.BlockSpec((512, 128), lambda i: (0, 0)),
            pl.BlockSpec((1, 512), lambda i: (0, 0)),
        ],
        out_specs=blk,
        out_shape=jax.ShapeDtypeStruct((npk, 128), f32),
    )(pk(agg1a), pk(agg1b), pk(dinvr), w1a_big, w1b_big, w2_big, b1t)
    p2 = p2p.reshape(np_, HALF)

    agg2a, agg2b = _agg_kernel(ep, np_, split_edges=True)(p2, p2, src2d, dst2d)

    outp = pl.pallas_call(
        _fin_body,
        grid=(grid,),
        in_specs=[blk, blk, blk, blk, pl.BlockSpec((1, 128), lambda i: (0, 0))],
        out_specs=blk,
        out_shape=jax.ShapeDtypeStruct((npk, 128), f32),
    )(pk(agg2a), pk(agg2b), p2p, pk(dinvr), b2t)

    return outp[: n // 8].reshape(n, HALF)[:, :out_f], kl[0, 0]
